# baseline (speedup 1.0000x reference)
import functools

import jax
import jax.numpy as jnp
from jax import lax
from jax.experimental import pallas as pl
from jax.experimental.pallas import tpu as pltpu
from jax.experimental.pallas import tpu_sc as plsc

K = 128
EDGE_TYPES = 16384
B, N = 4, 256
E = B * N * N
ROWS = E // 128

_NC = 2
_NS = 16
_L = 16
_NW = _NC * _NS

_NCHUNK = 2
_EC = E // _NCHUNK
_EPW = _EC // _NW

_RB = 256
_LOG2E = 1.4426950408889634


def _sc_fma_body(et_hbm, x_hbm, mul_hbm, bias_hbm, out_hbm,
                 idx_v, x_v, xx_v, mul_v, bias_v):
    wid = lax.axis_index("s") * _NC + lax.axis_index("c")
    base = wid * _EPW
    pltpu.sync_copy(et_hbm.at[pl.ds(base, _EPW)], idx_v)
    pltpu.sync_copy(x_hbm.at[pl.ds(base, _EPW)], x_v)
    pltpu.sync_copy(mul_hbm, mul_v)
    pltpu.sync_copy(bias_hbm, bias_v)

    def body(i, carry):
        s = pl.ds(i * _L, _L)
        idx = idx_v[s]
        m = plsc.load_gather(mul_v, [idx])
        bb = plsc.load_gather(bias_v, [idx])
        xx_v[s] = m * x_v[s] + bb
        return carry

    lax.fori_loop(0, _EPW // _L, body, 0)
    pltpu.sync_copy(xx_v, out_hbm.at[pl.ds(base, _EPW)])


@functools.cache
def _sc_fma():
    return pl.kernel(
        _sc_fma_body,
        mesh=plsc.VectorSubcoreMesh(core_axis_name="c", subcore_axis_name="s"),
        compiler_params=pltpu.CompilerParams(needs_layout_passes=False),
        out_type=jax.ShapeDtypeStruct((_EC,), jnp.float32),
        scratch_types=[
            pltpu.VMEM((_EPW,), jnp.int32),
            pltpu.VMEM((_EPW,), jnp.float32),
            pltpu.VMEM((_EPW,), jnp.float32),
            pltpu.VMEM((EDGE_TYPES,), jnp.float32),
            pltpu.VMEM((EDGE_TYPES,), jnp.float32),
        ],
    )


def _tc_rbf_body(mean_ref, temp_ref, xx_ref, out_ref):
    mean = mean_ref[0]
    ntemp = -jnp.abs(temp_ref[0]) * _LOG2E
    xx = xx_ref[...]
    d = xx[:, :, None] - mean[None, None, :]
    out_ref[...] = jnp.exp2(d * d * ntemp[None, None, :])


def _tc_rbf_first(xx_c, meanr, tempr):
    nblk = (ROWS // _NCHUNK) // _RB
    return pl.pallas_call(
        _tc_rbf_body,
        grid=(nblk,),
        in_specs=[
            pl.BlockSpec((1, K), lambda i: (0, 0)),
            pl.BlockSpec((1, K), lambda i: (0, 0)),
            pl.BlockSpec((_RB, 128), lambda i: (i, 0)),
        ],
        out_specs=pl.BlockSpec((_RB, 128, K), lambda i: (i, 0, 0)),
        out_shape=jax.ShapeDtypeStruct((ROWS, 128, K), jnp.float32),
    )(meanr, tempr, xx_c)


def _tc_rbf_second(xx_c, meanr, tempr, prev):
    nblk = (ROWS // _NCHUNK) // _RB

    def body(mean_ref, temp_ref, xx_ref, prev_ref, out_ref):
        del prev_ref
        _tc_rbf_body(mean_ref, temp_ref, xx_ref, out_ref)

    return pl.pallas_call(
        body,
        grid=(nblk,),
        in_specs=[
            pl.BlockSpec((1, K), lambda i: (0, 0)),
            pl.BlockSpec((1, K), lambda i: (0, 0)),
            pl.BlockSpec((_RB, 128), lambda i: (i, 0)),
            pl.BlockSpec(memory_space=pl.ANY),
        ],
        out_specs=pl.BlockSpec((_RB, 128, K), lambda i: (i + nblk, 0, 0)),
        out_shape=jax.ShapeDtypeStruct((ROWS, 128, K), jnp.float32),
        input_output_aliases={3: 0},
    )(meanr, tempr, xx_c, prev)


def kernel(x, edge_types, means, temps, mul_w, bias_w):
    et = edge_types.reshape(E).astype(jnp.int32)
    xf = x.reshape(E).astype(jnp.float32)
    mulf = mul_w.reshape(EDGE_TYPES)
    biasf = bias_w.reshape(EDGE_TYPES)
    meanr = means.reshape(1, K)
    tempr = temps.reshape(1, K)

    sc = _sc_fma()
    xx0 = sc(et[:_EC], xf[:_EC], mulf, biasf)
    xx1 = sc(et[_EC:], xf[_EC:], mulf, biasf)
    out = _tc_rbf_first(xx0.reshape(ROWS // 2, 128), meanr, tempr)
    out = _tc_rbf_second(xx1.reshape(ROWS // 2, 128), meanr, tempr, out)
    return out.reshape(B, N, N, K).astype(means.dtype)


# speedup vs baseline: 1.1602x; 1.1602x over previous
import functools

import jax
import jax.numpy as jnp
from jax import lax
from jax.experimental import pallas as pl
from jax.experimental.pallas import tpu as pltpu
from jax.experimental.pallas import tpu_sc as plsc

K = 128
EDGE_TYPES = 16384
B, N = 4, 256
E = B * N * N
ROWS = E // 128

_NC = 2
_NS = 16
_L = 16
_NW = _NC * _NS
_EPW = E // _NW

_RB = 256
_LOG2E = 1.4426950408889634


def _sc_fma_body(et_hbm, x_hbm, mul_hbm, bias_hbm, out_hbm,
                 idx_v, x_v, xx_v, mul_v, bias_v, sem):
    wid = lax.axis_index("s") * _NC + lax.axis_index("c")
    base = wid * _EPW
    c1 = pltpu.async_copy(et_hbm.at[pl.ds(base, _EPW)], idx_v, sem)
    c2 = pltpu.async_copy(x_hbm.at[pl.ds(base, _EPW)], x_v, sem)
    c3 = pltpu.async_copy(mul_hbm, mul_v, sem)
    c4 = pltpu.async_copy(bias_hbm, bias_v, sem)
    c1.wait()
    c2.wait()
    c3.wait()
    c4.wait()

    @plsc.parallel_loop(0, _EPW // _L, unroll=8)
    def _(i):
        s = pl.ds(i * _L, _L)
        idx = idx_v[s]
        m = plsc.load_gather(mul_v, [idx])
        bb = plsc.load_gather(bias_v, [idx])
        xx_v[s] = m * x_v[s] + bb

    pltpu.sync_copy(xx_v, out_hbm.at[pl.ds(base, _EPW)])


@functools.cache
def _sc_fma():
    return pl.kernel(
        _sc_fma_body,
        mesh=plsc.VectorSubcoreMesh(core_axis_name="c", subcore_axis_name="s"),
        compiler_params=pltpu.CompilerParams(needs_layout_passes=False),
        out_type=jax.ShapeDtypeStruct((E,), jnp.float32),
        scratch_types=[
            pltpu.VMEM((_EPW,), jnp.int32),
            pltpu.VMEM((_EPW,), jnp.float32),
            pltpu.VMEM((_EPW,), jnp.float32),
            pltpu.VMEM((EDGE_TYPES,), jnp.float32),
            pltpu.VMEM((EDGE_TYPES,), jnp.float32),
            pltpu.SemaphoreType.DMA,
        ],
    )


def _tc_rbf_body(mean_ref, temp_ref, xx_ref, out_ref):
    mean = mean_ref[0]
    ntemp = -jnp.abs(temp_ref[0]) * _LOG2E
    xx = xx_ref[...]
    d = xx[:, :, None] - mean[None, None, :]
    out_ref[...] = jnp.exp2(d * d * ntemp[None, None, :])


def _tc_rbf(xx2, meanr, tempr):
    return pl.pallas_call(
        _tc_rbf_body,
        grid=(ROWS // _RB,),
        in_specs=[
            pl.BlockSpec((1, K), lambda i: (0, 0)),
            pl.BlockSpec((1, K), lambda i: (0, 0)),
            pl.BlockSpec((_RB, 128), lambda i: (i, 0)),
        ],
        out_specs=pl.BlockSpec((_RB, 128, K), lambda i: (i, 0, 0)),
        out_shape=jax.ShapeDtypeStruct((ROWS, 128, K), jnp.float32),
    )(meanr, tempr, xx2)


def kernel(x, edge_types, means, temps, mul_w, bias_w):
    et = edge_types.reshape(E).astype(jnp.int32)
    xf = x.reshape(E).astype(jnp.float32)
    mulf = mul_w.reshape(EDGE_TYPES)
    biasf = bias_w.reshape(EDGE_TYPES)
    xx = _sc_fma()(et, xf, mulf, biasf)
    out = _tc_rbf(xx.reshape(ROWS, 128),
                  means.reshape(1, K), temps.reshape(1, K))
    return out.reshape(B, N, N, K).astype(means.dtype)
